# baseline (speedup 1.0000x reference)
import functools

import jax
import jax.numpy as jnp
from jax import lax
from jax.experimental import pallas as pl
from jax.experimental.pallas import tpu as pltpu
from jax.experimental.pallas import tpu_sc as plsc

_NC = 2
_NS = 16
_NW = _NC * _NS
_CH = 32
_VEC = 16


@functools.lru_cache(maxsize=None)
def _build_gather1(B, D, tiled):
    bw = B // _NW
    nchunk = bw // _CH
    mesh = plsc.VectorSubcoreMesh(core_axis_name="c", subcore_axis_name="s")
    params = None if tiled else pltpu.CompilerParams(use_tc_tiling_on_sc=False)

    @functools.partial(
        pl.kernel,
        out_type=jax.ShapeDtypeStruct((B, D), jnp.float32),
        mesh=mesh,
        scratch_types=[
            pltpu.VMEM((bw,), jnp.int32),
            pltpu.VMEM((_CH, D), jnp.float32),
            pltpu.SemaphoreType.DMA,
        ],
        compiler_params=params,
    )
    def gather_k(ids, tab, out, idx, v1, sem):
        wid = lax.axis_index("s") * _NC + lax.axis_index("c")
        base = wid * bw
        pltpu.sync_copy(ids.at[pl.ds(base, bw)], idx)

        def chunk(c, carry):
            cb = c * _CH
            cps = []
            for v0 in range(0, _CH, _VEC):
                v = idx[pl.ds(cb + v0, _VEC)]
                for g in range(_VEC):
                    r = v[g]
                    d = v0 + g
                    cps.append(pltpu.async_copy(
                        tab.at[pl.ds(r, 1)], v1.at[pl.ds(d, 1)], sem))
            for cp in cps:
                cp.wait()
            pltpu.sync_copy(v1, out.at[pl.ds(base + cb, _CH)])
            return carry

        lax.fori_loop(0, nchunk, chunk, 0)

    return gather_k


def _tc_body(ue, uemb, ie, iemb, wut, wub, bu, wit, wib, bi,
             usersT_o, itemsT_o, ratingsT_o):
    u = jnp.dot(ue[...], wut[...], preferred_element_type=jnp.float32)
    u = u + jnp.dot(uemb[...], wub[...], preferred_element_type=jnp.float32)
    u = u + bu[...]
    t = jnp.dot(ie[...], wit[...], preferred_element_type=jnp.float32)
    t = t + jnp.dot(iemb[...], wib[...], preferred_element_type=jnp.float32)
    t = t + bi[...]
    usersT_o[...] = u.T
    itemsT_o[...] = t.T
    ratingsT_o[...] = jnp.sum(u * t, axis=-1, keepdims=True).T


def kernel(user_ids, item_ids, user_encodings, item_encodings,
           user_embed, item_embed, user_fc_w, user_fc_b,
           item_fc_w, item_fc_b):
    B = user_ids.shape[0]
    UD = user_encodings.shape[1]
    ID = item_encodings.shape[1]
    ED = user_embed.shape[1]
    HID = user_fc_w.shape[1]

    uids = user_ids.astype(jnp.int32)
    iids = item_ids.astype(jnp.int32)

    uemb = _build_gather1(B, ED, False)(uids, user_embed)
    iemb = _build_gather1(B, ED, False)(iids, item_embed)
    ie = _build_gather1(B, ID, False)(iids, item_encodings)
    ue = _build_gather1(B, UD, True)(uids, user_encodings)

    wut, wub = user_fc_w[:UD], user_fc_w[UD:]
    wit, wib = item_fc_w[:ID], item_fc_w[ID:]
    bu = user_fc_b.reshape(1, HID)
    bi = item_fc_b.reshape(1, HID)

    BLK = 2048
    grid = (B // BLK,)
    full = lambda i: (0, 0)
    row = lambda i: (i, 0)
    col = lambda i: (0, i)
    usersT, itemsT, ratingsT = pl.pallas_call(
        _tc_body,
        grid=grid,
        in_specs=[
            pl.BlockSpec((BLK, UD), row),
            pl.BlockSpec((BLK, ED), row),
            pl.BlockSpec((BLK, ID), row),
            pl.BlockSpec((BLK, ED), row),
            pl.BlockSpec((UD, HID), full),
            pl.BlockSpec((ED, HID), full),
            pl.BlockSpec((1, HID), full),
            pl.BlockSpec((ID, HID), full),
            pl.BlockSpec((ED, HID), full),
            pl.BlockSpec((1, HID), full),
        ],
        out_specs=[
            pl.BlockSpec((HID, BLK), col),
            pl.BlockSpec((HID, BLK), col),
            pl.BlockSpec((1, BLK), col),
        ],
        out_shape=(
            jax.ShapeDtypeStruct((HID, B), jnp.float32),
            jax.ShapeDtypeStruct((HID, B), jnp.float32),
            jax.ShapeDtypeStruct((1, B), jnp.float32),
        ),
    )(ue, uemb, ie, iemb, wut, wub, bu, wit, wib, bi)

    return usersT.T, itemsT.T, ratingsT.reshape(B)


# speedup vs baseline: 1.9456x; 1.9456x over previous
import functools

import jax
import jax.numpy as jnp
from jax import lax
from jax.experimental import pallas as pl
from jax.experimental.pallas import tpu as pltpu
from jax.experimental.pallas import tpu_sc as plsc

_NC = 2
_NS = 16
_NW = _NC * _NS
_CH = 32
_VEC = 16


@functools.lru_cache(maxsize=None)
def _build_gather1(B, D, tiled):
    bw = B // _NW
    nchunk = bw // _CH
    mesh = plsc.VectorSubcoreMesh(core_axis_name="c", subcore_axis_name="s")
    params = None if tiled else pltpu.CompilerParams(use_tc_tiling_on_sc=False)

    @functools.partial(
        pl.kernel,
        out_type=jax.ShapeDtypeStruct((B, D), jnp.float32),
        mesh=mesh,
        scratch_types=[
            pltpu.VMEM((bw,), jnp.int32),
            pltpu.VMEM((_CH, D), jnp.float32),
            pltpu.SemaphoreType.DMA,
        ],
        compiler_params=params,
    )
    def gather_k(ids, tab, out, idx, v1, sem):
        wid = lax.axis_index("s") * _NC + lax.axis_index("c")
        base = wid * bw
        pltpu.sync_copy(ids.at[pl.ds(base, bw)], idx)

        def chunk(c, carry):
            cb = c * _CH
            cps = []
            for v0 in range(0, _CH, _VEC):
                v = idx[pl.ds(cb + v0, _VEC)]
                for g in range(_VEC):
                    r = v[g]
                    d = v0 + g
                    cps.append(pltpu.async_copy(
                        tab.at[pl.ds(r, 1)], v1.at[pl.ds(d, 1)], sem))
            for cp in cps:
                cp.wait()
            pltpu.sync_copy(v1, out.at[pl.ds(base + cb, _CH)])
            return carry

        lax.fori_loop(0, nchunk, chunk, 0)

    return gather_k


def _tc_body(ue, uemb, ie, iemb, wut, wub, bu, wit, wib, bi,
             usersT_o, itemsT_o, ratingsT_o):
    u = jnp.dot(ue[...], wut[...], preferred_element_type=jnp.float32)
    u = u + jnp.dot(uemb[...], wub[...], preferred_element_type=jnp.float32)
    u = u + bu[...]
    t = jnp.dot(ie[...], wit[...], preferred_element_type=jnp.float32)
    t = t + jnp.dot(iemb[...], wib[...], preferred_element_type=jnp.float32)
    t = t + bi[...]
    usersT_o[...] = u.T
    itemsT_o[...] = t.T
    ratingsT_o[...] = jnp.sum(u * t, axis=-1, keepdims=True).T


def kernel(user_ids, item_ids, user_encodings, item_encodings,
           user_embed, item_embed, user_fc_w, user_fc_b,
           item_fc_w, item_fc_b):
    B = user_ids.shape[0]
    UD = user_encodings.shape[1]
    ID = item_encodings.shape[1]
    ED = user_embed.shape[1]
    HID = user_fc_w.shape[1]

    uids = user_ids.astype(jnp.int32)
    iids = item_ids.astype(jnp.int32)

    iemb = _build_gather1(B, ED, False)(iids, item_embed)
    ie = _build_gather1(B, ID, False)(iids, item_encodings)
    uemb = _build_gather1(B, ED, True)(uids, user_embed)
    ue = _build_gather1(B, UD, True)(uids, user_encodings)

    wut, wub = user_fc_w[:UD], user_fc_w[UD:]
    wit, wib = item_fc_w[:ID], item_fc_w[ID:]
    bu = user_fc_b.reshape(1, HID)
    bi = item_fc_b.reshape(1, HID)

    BLK = 2048
    grid = (B // BLK,)
    full = lambda i: (0, 0)
    row = lambda i: (i, 0)
    col = lambda i: (0, i)
    usersT, itemsT, ratingsT = pl.pallas_call(
        _tc_body,
        grid=grid,
        in_specs=[
            pl.BlockSpec((BLK, UD), row),
            pl.BlockSpec((BLK, ED), row),
            pl.BlockSpec((BLK, ID), row),
            pl.BlockSpec((BLK, ED), row),
            pl.BlockSpec((UD, HID), full),
            pl.BlockSpec((ED, HID), full),
            pl.BlockSpec((1, HID), full),
            pl.BlockSpec((ID, HID), full),
            pl.BlockSpec((ED, HID), full),
            pl.BlockSpec((1, HID), full),
        ],
        out_specs=[
            pl.BlockSpec((HID, BLK), col),
            pl.BlockSpec((HID, BLK), col),
            pl.BlockSpec((1, BLK), col),
        ],
        out_shape=(
            jax.ShapeDtypeStruct((HID, B), jnp.float32),
            jax.ShapeDtypeStruct((HID, B), jnp.float32),
            jax.ShapeDtypeStruct((1, B), jnp.float32),
        ),
    )(ue, uemb, ie, iemb, wut, wub, bu, wit, wib, bi)

    return usersT.T, itemsT.T, ratingsT.reshape(B)
